# baseline (speedup 1.0000x reference)
import jax
import jax.numpy as jnp
from jax.experimental import pallas as pl

_B, _C, _H, _W = 4, 96, 224, 224
_N = _C * _H * _W
_LANES = 512
_ROWS = _N // _LANES
_K = 8
_RB = _ROWS // _K


def _mse_body(o_ref, m_ref, g_ref, out_ref):
    k = pl.program_id(1)

    @pl.when(k == 0)
    def _init():
        out_ref[...] = jnp.zeros_like(out_ref)

    o = o_ref[...]
    m = m_ref[...]
    g = g_ref[...]
    d = (o - g) * m
    s1v = jnp.sum(d * d, axis=(0, 1))
    s2v = jnp.sum(m, axis=(0, 1))
    out_ref[0, 0, :] += s1v
    out_ref[0, 1, :] += s2v


def _partial_sums(o3, m3, g3, interpret=False):
    spec = pl.BlockSpec((1, _RB, _LANES), lambda b, k: (b, k, 0))
    return pl.pallas_call(
        _mse_body,
        grid=(_B, _K),
        in_specs=[spec, spec, spec],
        out_specs=pl.BlockSpec((1, 2, _LANES), lambda b, k: (b, 0, 0)),
        out_shape=jax.ShapeDtypeStruct((_B, 2, _LANES), jnp.float32),
        interpret=interpret,
    )(o3, m3, g3)


def kernel(output, mask, ground_truth, normalizer):
    o3 = output.reshape(_B, _ROWS, _LANES)
    m3 = mask.reshape(_B, _ROWS, _LANES)
    g3 = ground_truth.reshape(_B, _ROWS, _LANES)
    part = _partial_sums(o3, m3, g3)
    sums = part.sum(axis=-1)
    loss = jnp.mean(sums[:, 0] / sums[:, 1])
    return (loss, output, ground_truth)


# speedup vs baseline: 1.0052x; 1.0052x over previous
import jax
import jax.numpy as jnp
from jax.experimental import pallas as pl

_B, _C, _H, _W = 4, 96, 224, 224
_N = _C * _H * _W
_LANES = 512
_ROWS = _N // _LANES
_K = 6
_S = 4
_RB = _ROWS // (_K * _S)


def _mse_body(*refs):
    out_ref = refs[-1]
    k = pl.program_id(1)

    @pl.when(k == 0)
    def _init():
        out_ref[...] = jnp.zeros_like(out_ref)

    s1v = jnp.zeros((_LANES,), jnp.float32)
    s2v = jnp.zeros((_LANES,), jnp.float32)
    for s in range(_S):
        o = refs[s][...]
        m = refs[_S + s][...]
        g = refs[2 * _S + s][...]
        d = (o - g) * m
        s1v += jnp.sum(d * d, axis=(0, 1))
        s2v += jnp.sum(m, axis=(0, 1))
    out_ref[0, 0, :] += s1v
    out_ref[0, 1, :] += s2v


def _partial_sums(o3, m3, g3, interpret=False):
    def shard_spec(s):
        return pl.BlockSpec((1, _RB, _LANES), lambda b, k, s=s: (b, k * _S + s, 0))

    specs = [shard_spec(s) for s in range(_S)]
    return pl.pallas_call(
        _mse_body,
        grid=(_B, _K),
        in_specs=specs * 3,
        out_specs=pl.BlockSpec((1, 2, _LANES), lambda b, k: (b, 0, 0)),
        out_shape=jax.ShapeDtypeStruct((_B, 2, _LANES), jnp.float32),
        interpret=interpret,
    )(*([o3] * _S + [m3] * _S + [g3] * _S))


def kernel(output, mask, ground_truth, normalizer):
    o3 = output.reshape(_B, _ROWS, _LANES)
    m3 = mask.reshape(_B, _ROWS, _LANES)
    g3 = ground_truth.reshape(_B, _ROWS, _LANES)
    part = _partial_sums(o3, m3, g3)
    sums = part.sum(axis=-1)
    loss = jnp.mean(sums[:, 0] / sums[:, 1])
    return (loss, output, ground_truth)


# speedup vs baseline: 2.6459x; 2.6323x over previous
import jax
import jax.numpy as jnp
from jax.experimental import pallas as pl

_B, _C, _H, _W = 4, 96, 224, 224
_K = 6
_CB = _C // _K


def _mse_body(o_ref, m_ref, g_ref, out_ref):
    k = pl.program_id(1)

    @pl.when(k == 0)
    def _init():
        out_ref[...] = jnp.zeros_like(out_ref)

    o = o_ref[...]
    m = m_ref[...]
    g = g_ref[...]
    d = (o - g) * m
    s1v = jnp.sum(d * d, axis=(0, 1, 2))
    s2v = jnp.sum(m, axis=(0, 1, 2))
    out_ref[0, 0, :] += s1v
    out_ref[0, 1, :] += s2v


def _partial_sums(o, m, g, interpret=False):
    spec = pl.BlockSpec((1, _CB, _H, _W), lambda b, k: (b, k, 0, 0))
    return pl.pallas_call(
        _mse_body,
        grid=(_B, _K),
        in_specs=[spec, spec, spec],
        out_specs=pl.BlockSpec((1, 2, _W), lambda b, k: (b, 0, 0)),
        out_shape=jax.ShapeDtypeStruct((_B, 2, _W), jnp.float32),
        interpret=interpret,
    )(o, m, g)


def kernel(output, mask, ground_truth, normalizer):
    part = _partial_sums(output, mask, ground_truth)
    sums = part.sum(axis=-1)
    loss = jnp.mean(sums[:, 0] / sums[:, 1])
    return (loss, output, ground_truth)
